# baseline (speedup 1.0000x reference)
import jax
import jax.numpy as jnp
from jax import lax
from jax.experimental import pallas as pl

BB = 16


def _posenc_block(ts_ref, emb_ref, out_ref):
    t = ts_ref[...]
    bb, hist = t.shape
    expand = emb_ref.shape[1]
    tk = t[:, :, None]
    tj = t[:, None, :]
    kk2 = lax.broadcasted_iota(jnp.int32, (hist, hist), 0)
    jj2 = lax.broadcasted_iota(jnp.int32, (hist, hist), 1)
    tri = (kk2 < jj2)[None]
    c = ((tk < tj) | ((tk <= tj) & tri)).astype(jnp.int32)
    rank = jnp.sum(c, axis=1)
    e = emb_ref[...]
    e_hi = e.astype(jnp.bfloat16)
    e_lo = (e - e_hi.astype(jnp.float32)).astype(jnp.bfloat16)
    e2 = jnp.concatenate([e_hi, e_lo], axis=1)
    hq = hist // 4
    rnk = rank[:, None, :]
    ii4 = lax.broadcasted_iota(jnp.int32, (bb, hq, hist), 1) * 4
    outs = []
    for il in range(4):
        m_il = (rnk == ii4 + il).astype(jnp.bfloat16).reshape(bb * hq, hist)
        o2 = jnp.dot(m_il, e2, preferred_element_type=jnp.float32)
        outs.append(o2[:, :expand] + o2[:, expand:])
    out = jnp.concatenate(outs, axis=1)
    out_ref[...] = out.reshape(bb, hq, 4 * expand)


def kernel(ts, pos_embeddings):
    batch, hist = ts.shape
    seq_len, expand = pos_embeddings.shape
    flat = pl.pallas_call(
        _posenc_block,
        grid=(batch // BB,),
        in_specs=[
            pl.BlockSpec((BB, hist), lambda i: (i, 0)),
            pl.BlockSpec((seq_len, expand), lambda i: (0, 0)),
        ],
        out_specs=pl.BlockSpec((BB, hist // 4, 4 * expand), lambda i: (i, 0, 0)),
        out_shape=jax.ShapeDtypeStruct((batch, hist // 4, 4 * expand),
                                       jnp.float32),
    )(ts, pos_embeddings)
    return flat.reshape(batch, hist, expand)
